# baseline (speedup 1.0000x reference)
import functools

import jax
import jax.numpy as jnp
from jax import lax
from jax.experimental import pallas as pl
from jax.experimental.pallas import tpu as pltpu
from jax.experimental.pallas import tpu_sc as plsc

N_NODES = 10000
N_EDGES = 320000
NE2 = N_EDGES // 2
F = 64
W128 = 128
EMB = 12

NC = 2
NS = 16
NW = NC * NS
EPW = N_EDGES // NW
KB = 80
PB = KB // 2
NB = EPW // KB
RPT = 624
TAIL = N_NODES - RPT * NS

NODE_BLK = 1000
EDGE_BLK = 2000


def _silu(y):
    return y * jax.nn.sigmoid(y)


def _dot(a, w):
    return jnp.dot(a, w, preferred_element_type=jnp.float32)


def _ln(y, g, b):
    a = jnp.full((F, 1), 1.0 / F, jnp.float32)
    mu = _dot(y, a)
    m2 = _dot(y * y, a)
    inv = jax.lax.rsqrt(jnp.maximum(m2 - mu * mu, 0.0) + 1e-5)
    return (y - mu) * (inv * g) + b


def _ln2(y, g2, b2, a2, m2b):
    mu2 = _dot(y, a2)
    s2 = _dot(y * y, a2)
    inv2 = jax.lax.rsqrt(jnp.maximum(s2 - mu2 * mu2, 0.0) + 1e-5)
    p = _dot(inv2, m2b)
    q = _dot(mu2 * inv2, m2b)
    return (y * p - q) * g2 + b2


def _node_prep_body(xp, wn1, bn1, gn1, hn1, wn2, bn2, gn2, hn2, wea, bea, web,
                    nodes_o, u_o):
    xb = xp[...]
    h = _silu(_ln(_dot(xb, wn1[...]) + bn1[...], gn1[...], hn1[...]))
    nodes_o[...] = _silu(_ln(_dot(h, wn2[...]) + bn2[...], gn2[...], hn2[...]))
    t1 = _dot(xb, wea[...]) + bea[...]
    t2 = _dot(xb, web[...])
    u_o[...] = jnp.concatenate([t1, t2], axis=-1)


def _edge_encode_body(preb, a2, m2b, gl1, hl1, w2, b2, gl2, hl2, out):
    h = _silu(_ln2(preb[...], gl1[...], hl1[...], a2[...], m2b[...]))
    out[...] = _silu(_ln2(_dot(h, w2[...]) + b2[...], gl2[...], hl2[...],
                          a2[...], m2b[...]))


def _node_cell_body(nodes, pmsg, wn, wm, b1, gl1, hl1, w2, b2, gl2, hl2,
                    ws, bs, wd, nodes_o, u_o):
    nb = nodes[...]
    msg = (pmsg[0] + pmsg[1])[..., :F]
    h = _silu(_ln(_dot(nb, wn[...]) + _dot(msg, wm[...]) + b1[...],
                  gl1[...], hl1[...]))
    nn = _silu(_ln(_dot(h, w2[...]) + b2[...], gl2[...], hl2[...])) + nb
    nodes_o[...] = nn
    t1 = _dot(nn, ws[...]) + bs[...]
    t2 = _dot(nn, wd[...])
    u_o[...] = jnp.concatenate([t1, t2], axis=-1)


def _edge_cell_body(preb, e, a2, m2b, we, gl1, hl1, w2, b2, gl2, hl2, out):
    eb = e[...]
    y = preb[...] + _dot(eb, we[...])
    h = _silu(_ln2(y, gl1[...], hl1[...], a2[...], m2b[...]))
    out[...] = _silu(_ln2(_dot(h, w2[...]) + b2[...], gl2[...], hl2[...],
                          a2[...], m2b[...])) + eb


def _out_head_body(nodes, w1, b1, gl, hl, w2, b2, out):
    h = jnp.tanh(_ln(_dot(nodes[...], w1[...]) + b1[...], gl[...], hl[...]))
    emb = _dot(h, w2[...]) + b2[...]
    nrm = jnp.sqrt(jnp.sum(emb * emb, axis=-1, keepdims=True))
    out[...] = emb / jnp.maximum(nrm, 1e-12)


def _rows(blk, width=F):
    return pl.BlockSpec((blk, width), lambda i: (i, 0))


def _full(shape):
    nd = len(shape)
    return pl.BlockSpec(shape, lambda i: (0,) * nd)


def _node_prep(xp, args):
    in_specs = [pl.BlockSpec((NODE_BLK, 8), lambda i: (i, 0))]
    in_specs += [_full(a.shape) for a in args]
    return pl.pallas_call(
        _node_prep_body,
        grid=(N_NODES // NODE_BLK,),
        in_specs=in_specs,
        out_specs=[_rows(NODE_BLK, F), _rows(NODE_BLK, W128)],
        out_shape=[jax.ShapeDtypeStruct((N_NODES, F), jnp.float32),
                   jax.ShapeDtypeStruct((N_NODES, W128), jnp.float32)],
    )(xp, *args)


def _edge_encode(pre, args):
    in_specs = [_rows(EDGE_BLK, W128)] + [_full(a.shape) for a in args]
    return pl.pallas_call(
        _edge_encode_body,
        grid=(NE2 // EDGE_BLK,),
        in_specs=in_specs,
        out_specs=_rows(EDGE_BLK, W128),
        out_shape=jax.ShapeDtypeStruct((NE2, W128), jnp.float32),
    )(pre, *args)


def _node_cell(nodes, pmsg, args):
    in_specs = [_rows(NODE_BLK, F),
                pl.BlockSpec((2, NODE_BLK, W128), lambda i: (0, i, 0))]
    in_specs += [_full(a.shape) for a in args]
    return pl.pallas_call(
        _node_cell_body,
        grid=(N_NODES // NODE_BLK,),
        in_specs=in_specs,
        out_specs=[_rows(NODE_BLK, F), _rows(NODE_BLK, W128)],
        out_shape=[jax.ShapeDtypeStruct((N_NODES, F), jnp.float32),
                   jax.ShapeDtypeStruct((N_NODES, W128), jnp.float32)],
    )(nodes, pmsg, *args)


def _edge_cell(pre, edges, args):
    in_specs = ([_rows(EDGE_BLK, W128), _rows(EDGE_BLK, W128)] +
                [_full(a.shape) for a in args])
    return pl.pallas_call(
        _edge_cell_body,
        grid=(NE2 // EDGE_BLK,),
        in_specs=in_specs,
        out_specs=_rows(EDGE_BLK, W128),
        out_shape=jax.ShapeDtypeStruct((NE2, W128), jnp.float32),
    )(pre, edges, *args)


def _out_head(nodes, args):
    in_specs = [_rows(NODE_BLK, F)] + [_full(a.shape) for a in args]
    return pl.pallas_call(
        _out_head_body,
        grid=(N_NODES // NODE_BLK,),
        in_specs=in_specs,
        out_specs=pl.BlockSpec((NODE_BLK, W128), lambda i: (i, 0)),
        out_shape=jax.ShapeDtypeStruct((N_NODES, W128), jnp.float32),
    )(nodes, *args)


@functools.lru_cache(maxsize=None)
def _mesh():
    return plsc.VectorSubcoreMesh(core_axis_name="c", subcore_axis_name="s")


def _sc_gather_body(u, srcI, dstI, pre,
                    ibS, ibD, gbS, gbD, pb, semg0, semg1, semw0, semw1):
    wid = lax.axis_index("s") * NC + lax.axis_index("c")
    pltpu.sync_copy(srcI.at[wid], ibS)
    pltpu.sync_copy(dstI.at[wid], ibD)
    pbase0 = wid * (EPW // 2)
    semg = (semg0, semg1)
    semw = (semw0, semw1)

    def fire_gathers(b, s):
        pltpu.async_copy(u.at[ibS.at[b]], gbS.at[s], semg[s])
        pltpu.async_copy(u.at[ibD.at[b]], gbD.at[s], semg[s])

    def compute_pre(s):
        def pair(m, c):
            for k in range(4):
                lo = pl.ds(k * 16, 16)
                hi = pl.ds(F + k * 16, 16)
                pb[s, m, lo] = gbS[s, 2 * m, lo] + gbD[s, 2 * m, hi]
                pb[s, m, hi] = gbS[s, 2 * m + 1, lo] + gbD[s, 2 * m + 1, hi]
            return c
        lax.fori_loop(0, PB, pair, 0)

    def process(b, s):
        @pl.when(b < NB)
        def _():
            pltpu.make_async_copy(u.at[pl.ds(0, KB)], gbS.at[s],
                                  semg[s]).wait()
            pltpu.make_async_copy(u.at[pl.ds(0, KB)], gbD.at[s],
                                  semg[s]).wait()
            compute_pre(s)
            wd = pltpu.async_copy(pb.at[s], pre.at[pl.ds(pbase0 + b * PB, PB)],
                                  semw[s])
            wd.wait()

            @pl.when(b + 2 < NB)
            def _():
                fire_gathers(b + 2, s)

    fire_gathers(0, 0)
    fire_gathers(1, 1)

    def step(m, c):
        process(2 * m, 0)
        process(2 * m + 1, 1)
        return c

    lax.fori_loop(0, (NB + 2) // 2, step, 0)


@functools.lru_cache(maxsize=None)
def _sc_gather_kernel():
    return pl.kernel(
        _sc_gather_body,
        out_type=jax.ShapeDtypeStruct((NE2, W128), jnp.float32),
        mesh=_mesh(),
        scratch_types=[
            pltpu.VMEM((NB, KB), jnp.int32),
            pltpu.VMEM((NB, KB), jnp.int32),
            pltpu.VMEM((2, KB, W128), jnp.float32),
            pltpu.VMEM((2, KB, W128), jnp.float32),
            pltpu.VMEM((2, PB, W128), jnp.float32),
            pltpu.SemaphoreType.DMA,
            pltpu.SemaphoreType.DMA,
            pltpu.SemaphoreType.DMA,
            pltpu.SemaphoreType.DMA,
        ],
    )


def _sc_gather(u, srcI, dstI):
    return _sc_gather_kernel()(u, srcI, dstI)


def _sc_scatter_body(edges, src1, dst1, zrows, out,
                     acc, ibS, ibD, pbf, eb, seml0, seml1, sems0, sems1):
    cid = lax.axis_index("c")
    sid = lax.axis_index("s")
    wid = sid * NC + cid
    pltpu.sync_copy(zrows.at[pl.ds(sid * RPT, RPT)],
                    acc.at[pl.ds(sid * RPT, RPT)])

    @pl.when(sid == 0)
    def _():
        pltpu.sync_copy(zrows.at[pl.ds(RPT * NS, TAIL)],
                        acc.at[pl.ds(RPT * NS, TAIL)])

    z16 = jnp.zeros((16,), jnp.float32)

    def zrow(i, c):
        for k in range(4):
            eb[0, i, pl.ds(F + k * 16, 16)] = z16
            eb[1, i, pl.ds(F + k * 16, 16)] = z16
        return c

    lax.fori_loop(0, KB, zrow, 0)
    plsc.subcore_barrier()
    base0 = wid * EPW
    pbase0 = wid * (EPW // 2)
    seml = (seml0, seml1)
    sems = (sems0, sems1)

    def fire_load(b, s):
        base = base0 + b * KB
        pltpu.async_copy(edges.at[pl.ds(pbase0 + b * PB, PB)], pbf.at[s],
                         seml[s])
        pltpu.async_copy(src1.at[pl.ds(base, KB)], ibS.at[s], seml[s])
        pltpu.async_copy(dst1.at[pl.ds(base, KB)], ibD.at[s], seml[s])

    def unpack(s):
        def pair(m, c):
            for k in range(4):
                lo = pl.ds(k * 16, 16)
                hi = pl.ds(F + k * 16, 16)
                eb[s, 2 * m, lo] = pbf[s, m, lo]
                eb[s, 2 * m + 1, lo] = pbf[s, m, hi]
            return c
        lax.fori_loop(0, PB, pair, 0)

    def process(b, s):
        @pl.when(b < NB)
        def _():
            pltpu.make_async_copy(edges.at[pl.ds(0, PB)], pbf.at[s],
                                  seml[s]).wait()
            pltpu.make_async_copy(src1.at[pl.ds(0, KB)], ibS.at[s],
                                  seml[s]).wait()
            pltpu.make_async_copy(dst1.at[pl.ds(0, KB)], ibD.at[s],
                                  seml[s]).wait()
            unpack(s)
            d1 = pltpu.async_copy(eb.at[s], acc.at[ibD.at[s]], sems[s],
                                  add=True)
            d2 = pltpu.async_copy(eb.at[s], acc.at[ibS.at[s]], sems[s],
                                  add=True)
            d1.wait()
            d2.wait()

            @pl.when(b + 2 < NB)
            def _():
                fire_load(b + 2, s)

    fire_load(0, 0)
    fire_load(1, 1)

    def step(m, c):
        process(2 * m, 0)
        process(2 * m + 1, 1)
        return c

    lax.fori_loop(0, (NB + 2) // 2, step, 0)
    plsc.subcore_barrier()
    pltpu.sync_copy(acc.at[pl.ds(sid * RPT, RPT)],
                    out.at[cid, pl.ds(sid * RPT, RPT)])

    @pl.when(sid == 0)
    def _():
        pltpu.sync_copy(acc.at[pl.ds(RPT * NS, TAIL)],
                        out.at[cid, pl.ds(RPT * NS, TAIL)])


@functools.lru_cache(maxsize=None)
def _sc_scatter_kernel():
    return pl.kernel(
        _sc_scatter_body,
        out_type=jax.ShapeDtypeStruct((NC, N_NODES, W128), jnp.float32),
        mesh=_mesh(),
        scratch_types=[
            pltpu.VMEM_SHARED((N_NODES, W128), jnp.float32),
            pltpu.VMEM((2, KB), jnp.int32),
            pltpu.VMEM((2, KB), jnp.int32),
            pltpu.VMEM((2, PB, W128), jnp.float32),
            pltpu.VMEM((2, KB, W128), jnp.float32),
            pltpu.SemaphoreType.DMA,
            pltpu.SemaphoreType.DMA,
            pltpu.SemaphoreType.DMA,
            pltpu.SemaphoreType.DMA,
        ],
    )


def _sc_scatter(edges, src1, dst1, zrows):
    return _sc_scatter_kernel()(edges, src1, dst1, zrows)


def _rowvec(v):
    return v.reshape(1, -1)


def _rowvec2(v):
    return jnp.concatenate([v, v]).reshape(1, -1)


def _blkdiag(w):
    z = jnp.zeros_like(w)
    return jnp.block([[w, z], [z, w]])


def _lnp(layer):
    return [_rowvec(layer["ln_g"]), _rowvec(layer["ln_b"])]


def _lnp2(layer):
    return [_rowvec2(layer["ln_g"]), _rowvec2(layer["ln_b"])]


def kernel(x, graph, params):
    src = graph[0]
    dst = graph[1]
    srcI = src.reshape(NW, NB, KB)
    dstI = dst.reshape(NW, NB, KB)
    ne = params["node_encoder"]
    ee = params["edge_encoder"]
    cells = params["cells"]
    ol = params["output_layer"]

    xp = jnp.pad(x, ((0, 0), (0, 5)))
    wn1 = jnp.pad(ne[0]["W"], ((0, 5), (0, 0)))
    we1 = ee[0]["W"]
    wea = jnp.pad(we1[:3], ((0, 5), (0, 0)))
    web = jnp.pad(we1[3:], ((0, 5), (0, 0)))

    half = (jnp.arange(W128) >= F).astype(jnp.float32)
    a2 = jnp.stack([(1.0 - half) / F, half / F], axis=1)
    m2b = jnp.stack([1.0 - half, half], axis=0)

    prep_args = ([wn1, _rowvec(ne[0]["b"])] + _lnp(ne[0]) +
                 [ne[1]["W"], _rowvec(ne[1]["b"])] + _lnp(ne[1]) +
                 [wea, _rowvec(ee[0]["b"]), web])
    nodes, u = _node_prep(xp, prep_args)

    pre = _sc_gather(u, srcI, dstI)
    enc_args = ([a2, m2b] + _lnp2(ee[0]) +
                [_blkdiag(ee[1]["W"]), _rowvec2(ee[1]["b"])] + _lnp2(ee[1]))
    edges = _edge_encode(pre, enc_args)

    zrows = jnp.zeros((N_NODES, W128), jnp.float32)
    for cell in cells:
        nn0, nn1 = cell["node_network"]
        en0, en1 = cell["edge_network"]
        pmsg = _sc_scatter(edges, src, dst, zrows)
        cell_args = ([nn0["W"][:F], nn0["W"][F:], _rowvec(nn0["b"])] +
                     _lnp(nn0) +
                     [nn1["W"], _rowvec(nn1["b"])] + _lnp(nn1) +
                     [en0["W"][:F], _rowvec(en0["b"]), en0["W"][F:2 * F]])
        nodes, u = _node_cell(nodes, pmsg, cell_args)
        pre = _sc_gather(u, srcI, dstI)
        edge_args = ([a2, m2b, _blkdiag(en0["W"][2 * F:])] + _lnp2(en0) +
                     [_blkdiag(en1["W"]), _rowvec2(en1["b"])] + _lnp2(en1))
        edges = _edge_cell(pre, edges, edge_args)

    w2p = jnp.pad(ol[1]["W"], ((0, 0), (0, 128 - EMB)))
    b2p = jnp.pad(ol[1]["b"], ((0, 128 - EMB)))
    head_args = ([ol[0]["W"], _rowvec(ol[0]["b"])] + _lnp(ol[0]) +
                 [w2p, _rowvec(b2p)])
    embp = _out_head(nodes, head_args)
    return embp[:, :EMB], nodes, edges.reshape(N_EDGES, F)


# speedup vs baseline: 1.1817x; 1.1817x over previous
import functools

import jax
import jax.numpy as jnp
from jax import lax
from jax.experimental import pallas as pl
from jax.experimental.pallas import tpu as pltpu
from jax.experimental.pallas import tpu_sc as plsc

N_NODES = 10000
N_EDGES = 320000
NE2 = N_EDGES // 2
F = 64
W128 = 128
EMB = 12

NC = 2
NS = 16
NW = NC * NS
EPW = N_EDGES // NW
KB = 80
PB = KB // 2
NB = EPW // KB
RPT = 624
TAIL = N_NODES - RPT * NS

NODE_BLK = 1000
EDGE_BLK = 2000


def _silu(y):
    return y * jax.nn.sigmoid(y)


def _dot(a, w):
    return jnp.dot(a, w, preferred_element_type=jnp.float32)


def _ln(y, g, b):
    a = jnp.full((F, 1), 1.0 / F, jnp.float32)
    mu = _dot(y, a)
    m2 = _dot(y * y, a)
    inv = jax.lax.rsqrt(jnp.maximum(m2 - mu * mu, 0.0) + 1e-5)
    return (y - mu) * (inv * g) + b


def _ln2(y, g2, b2, a2, m2b):
    mu2 = _dot(y, a2)
    s2 = _dot(y * y, a2)
    inv2 = jax.lax.rsqrt(jnp.maximum(s2 - mu2 * mu2, 0.0) + 1e-5)
    p = _dot(inv2, m2b)
    q = _dot(mu2 * inv2, m2b)
    return (y * p - q) * g2 + b2


def _node_prep_body(xp, wn1, bn1, gn1, hn1, wn2, bn2, gn2, hn2, wea, bea, web,
                    nodes_o, u_o):
    xb = xp[...]
    h = _silu(_ln(_dot(xb, wn1[...]) + bn1[...], gn1[...], hn1[...]))
    nodes_o[...] = _silu(_ln(_dot(h, wn2[...]) + bn2[...], gn2[...], hn2[...]))
    t1 = _dot(xb, wea[...]) + bea[...]
    t2 = _dot(xb, web[...])
    u_o[...] = jnp.concatenate([t1, t2], axis=-1)


def _edge_encode_body(preb, a2, m2b, gl1, hl1, w2, b2, gl2, hl2, out):
    h = _silu(_ln2(preb[...], gl1[...], hl1[...], a2[...], m2b[...]))
    out[...] = _silu(_ln2(_dot(h, w2[...]) + b2[...], gl2[...], hl2[...],
                          a2[...], m2b[...]))


def _node_cell_body(nodes, pmsg, wn, wm, b1, gl1, hl1, w2, b2, gl2, hl2,
                    ws, bs, wd, nodes_o, u_o):
    nb = nodes[...]
    msg = (pmsg[0] + pmsg[1])[..., :F]
    h = _silu(_ln(_dot(nb, wn[...]) + _dot(msg, wm[...]) + b1[...],
                  gl1[...], hl1[...]))
    nn = _silu(_ln(_dot(h, w2[...]) + b2[...], gl2[...], hl2[...])) + nb
    nodes_o[...] = nn
    t1 = _dot(nn, ws[...]) + bs[...]
    t2 = _dot(nn, wd[...])
    u_o[...] = jnp.concatenate([t1, t2], axis=-1)


def _edge_cell_body(preb, e, a2, m2b, we, gl1, hl1, w2, b2, gl2, hl2, out):
    eb = e[...]
    y = preb[...] + _dot(eb, we[...])
    h = _silu(_ln2(y, gl1[...], hl1[...], a2[...], m2b[...]))
    out[...] = _silu(_ln2(_dot(h, w2[...]) + b2[...], gl2[...], hl2[...],
                          a2[...], m2b[...])) + eb


def _out_head_body(nodes, w1, b1, gl, hl, w2, b2, out):
    h = jnp.tanh(_ln(_dot(nodes[...], w1[...]) + b1[...], gl[...], hl[...]))
    emb = _dot(h, w2[...]) + b2[...]
    nrm = jnp.sqrt(jnp.sum(emb * emb, axis=-1, keepdims=True))
    out[...] = emb / jnp.maximum(nrm, 1e-12)


def _rows(blk, width=F):
    return pl.BlockSpec((blk, width), lambda i: (i, 0))


def _full(shape):
    nd = len(shape)
    return pl.BlockSpec(shape, lambda i: (0,) * nd)


def _node_prep(xp, args):
    in_specs = [pl.BlockSpec((NODE_BLK, 8), lambda i: (i, 0))]
    in_specs += [_full(a.shape) for a in args]
    return pl.pallas_call(
        _node_prep_body,
        grid=(N_NODES // NODE_BLK,),
        in_specs=in_specs,
        out_specs=[_rows(NODE_BLK, F), _rows(NODE_BLK, W128)],
        out_shape=[jax.ShapeDtypeStruct((N_NODES, F), jnp.float32),
                   jax.ShapeDtypeStruct((N_NODES, W128), jnp.float32)],
    )(xp, *args)


def _edge_encode(pre, args):
    in_specs = [_rows(EDGE_BLK, W128)] + [_full(a.shape) for a in args]
    return pl.pallas_call(
        _edge_encode_body,
        grid=(NE2 // EDGE_BLK,),
        in_specs=in_specs,
        out_specs=_rows(EDGE_BLK, W128),
        out_shape=jax.ShapeDtypeStruct((NE2, W128), jnp.float32),
    )(pre, *args)


def _node_cell(nodes, pmsg, args):
    in_specs = [_rows(NODE_BLK, F),
                pl.BlockSpec((2, NODE_BLK, W128), lambda i: (0, i, 0))]
    in_specs += [_full(a.shape) for a in args]
    return pl.pallas_call(
        _node_cell_body,
        grid=(N_NODES // NODE_BLK,),
        in_specs=in_specs,
        out_specs=[_rows(NODE_BLK, F), _rows(NODE_BLK, W128)],
        out_shape=[jax.ShapeDtypeStruct((N_NODES, F), jnp.float32),
                   jax.ShapeDtypeStruct((N_NODES, W128), jnp.float32)],
    )(nodes, pmsg, *args)


def _edge_cell(pre, edges, args):
    in_specs = ([_rows(EDGE_BLK, W128), _rows(EDGE_BLK, W128)] +
                [_full(a.shape) for a in args])
    return pl.pallas_call(
        _edge_cell_body,
        grid=(NE2 // EDGE_BLK,),
        in_specs=in_specs,
        out_specs=_rows(EDGE_BLK, W128),
        out_shape=jax.ShapeDtypeStruct((NE2, W128), jnp.float32),
    )(pre, edges, *args)


def _out_head(nodes, args):
    in_specs = [_rows(NODE_BLK, F)] + [_full(a.shape) for a in args]
    return pl.pallas_call(
        _out_head_body,
        grid=(N_NODES // NODE_BLK,),
        in_specs=in_specs,
        out_specs=pl.BlockSpec((NODE_BLK, W128), lambda i: (i, 0)),
        out_shape=jax.ShapeDtypeStruct((N_NODES, W128), jnp.float32),
    )(nodes, *args)


@functools.lru_cache(maxsize=None)
def _mesh():
    return plsc.VectorSubcoreMesh(core_axis_name="c", subcore_axis_name="s")


def _sc_gather_body(u, srcI, dstI, pre,
                    ibS, ibD, gbS, gbD, pb, semg0, semg1, semw0, semw1):
    wid = lax.axis_index("s") * NC + lax.axis_index("c")
    pltpu.sync_copy(srcI.at[wid], ibS)
    pltpu.sync_copy(dstI.at[wid], ibD)
    pbase0 = wid * (EPW // 2)
    semg = (semg0, semg1)
    semw = (semw0, semw1)

    def fire_gathers(b, s):
        pltpu.async_copy(u.at[ibS.at[b]], gbS.at[s], semg[s])
        pltpu.async_copy(u.at[ibD.at[b]], gbD.at[s], semg[s])

    def compute_pre(s):
        def quad(j, c):
            for mm in range(4):
                m = 4 * j + mm
                for k in range(4):
                    lo = pl.ds(k * 16, 16)
                    hi = pl.ds(F + k * 16, 16)
                    pb[s, m, lo] = gbS[s, 2 * m, lo] + gbD[s, 2 * m, hi]
                    pb[s, m, hi] = (gbS[s, 2 * m + 1, lo] +
                                    gbD[s, 2 * m + 1, hi])
            return c
        lax.fori_loop(0, PB // 4, quad, 0)

    def process(b, s):
        @pl.when(b < NB)
        def _():
            @pl.when(b >= 2)
            def _():
                pltpu.make_async_copy(pre.at[pl.ds(0, PB)], pb.at[s],
                                      semw[s]).wait()
            pltpu.make_async_copy(u.at[pl.ds(0, KB)], gbS.at[s],
                                  semg[s]).wait()
            pltpu.make_async_copy(u.at[pl.ds(0, KB)], gbD.at[s],
                                  semg[s]).wait()
            compute_pre(s)
            pltpu.async_copy(pb.at[s], pre.at[pl.ds(pbase0 + b * PB, PB)],
                             semw[s])

            @pl.when(b + 2 < NB)
            def _():
                fire_gathers(b + 2, s)

            @pl.when(b + 2 >= NB)
            def _():
                pltpu.make_async_copy(pre.at[pl.ds(0, PB)], pb.at[s],
                                      semw[s]).wait()

    fire_gathers(0, 0)
    fire_gathers(1, 1)

    def step(m, c):
        process(2 * m, 0)
        process(2 * m + 1, 1)
        return c

    lax.fori_loop(0, (NB + 2) // 2, step, 0)


@functools.lru_cache(maxsize=None)
def _sc_gather_kernel():
    return pl.kernel(
        _sc_gather_body,
        out_type=jax.ShapeDtypeStruct((NE2, W128), jnp.float32),
        mesh=_mesh(),
        scratch_types=[
            pltpu.VMEM((NB, KB), jnp.int32),
            pltpu.VMEM((NB, KB), jnp.int32),
            pltpu.VMEM((2, KB, W128), jnp.float32),
            pltpu.VMEM((2, KB, W128), jnp.float32),
            pltpu.VMEM((2, PB, W128), jnp.float32),
            pltpu.SemaphoreType.DMA,
            pltpu.SemaphoreType.DMA,
            pltpu.SemaphoreType.DMA,
            pltpu.SemaphoreType.DMA,
        ],
    )


def _sc_gather(u, srcI, dstI):
    return _sc_gather_kernel()(u, srcI, dstI)


def _sc_scatter_body(edges, src1, dst1, zrows, out,
                     acc, ibS, ibD, pbf, eb, seml0, seml1, sems0, sems1):
    cid = lax.axis_index("c")
    sid = lax.axis_index("s")
    wid = sid * NC + cid
    pltpu.sync_copy(zrows.at[pl.ds(sid * RPT, RPT)],
                    acc.at[pl.ds(sid * RPT, RPT)])

    @pl.when(sid == 0)
    def _():
        pltpu.sync_copy(zrows.at[pl.ds(RPT * NS, TAIL)],
                        acc.at[pl.ds(RPT * NS, TAIL)])

    z16 = jnp.zeros((16,), jnp.float32)

    def zrow(i, c):
        for k in range(4):
            eb[0, i, pl.ds(F + k * 16, 16)] = z16
            eb[1, i, pl.ds(F + k * 16, 16)] = z16
        return c

    lax.fori_loop(0, KB, zrow, 0)
    plsc.subcore_barrier()
    base0 = wid * EPW
    pbase0 = wid * (EPW // 2)
    seml = (seml0, seml1)
    sems = (sems0, sems1)

    def fire_load(b, il, s):
        base = base0 + b * KB
        pltpu.async_copy(edges.at[pl.ds(pbase0 + b * PB, PB)], pbf.at[s],
                         seml[s])
        pltpu.async_copy(src1.at[pl.ds(base, KB)], ibS.at[il], seml[s])
        pltpu.async_copy(dst1.at[pl.ds(base, KB)], ibD.at[il], seml[s])

    def unpack(s):
        def quad(j, c):
            for mm in range(4):
                m = 4 * j + mm
                for k in range(4):
                    lo = pl.ds(k * 16, 16)
                    hi = pl.ds(F + k * 16, 16)
                    eb[s, 2 * m, lo] = pbf[s, m, lo]
                    eb[s, 2 * m + 1, lo] = pbf[s, m, hi]
            return c
        lax.fori_loop(0, PB // 4, quad, 0)

    def drain_scatters(s):
        pltpu.make_async_copy(edges.at[pl.ds(0, KB)], eb.at[s],
                              sems[s]).wait()
        pltpu.make_async_copy(edges.at[pl.ds(0, KB)], eb.at[s],
                              sems[s]).wait()

    def process(b, il, s):
        @pl.when(b < NB)
        def _():
            @pl.when(b >= 2)
            def _():
                drain_scatters(s)
            pltpu.make_async_copy(edges.at[pl.ds(0, PB)], pbf.at[s],
                                  seml[s]).wait()
            pltpu.make_async_copy(src1.at[pl.ds(0, KB)], ibS.at[il],
                                  seml[s]).wait()
            pltpu.make_async_copy(dst1.at[pl.ds(0, KB)], ibD.at[il],
                                  seml[s]).wait()
            unpack(s)
            pltpu.async_copy(eb.at[s], acc.at[ibD.at[il]], sems[s], add=True)
            pltpu.async_copy(eb.at[s], acc.at[ibS.at[il]], sems[s], add=True)

            @pl.when(b + 2 < NB)
            def _():
                fire_load(b + 2, (il + 2) % 4, s)

            @pl.when(b + 2 >= NB)
            def _():
                drain_scatters(s)

    fire_load(0, 0, 0)
    fire_load(1, 1, 1)

    def step(m, c):
        b = 4 * m
        process(b, 0, 0)
        process(b + 1, 1, 1)
        process(b + 2, 2, 0)
        process(b + 3, 3, 1)
        return c

    lax.fori_loop(0, (NB + 3) // 4, step, 0)
    plsc.subcore_barrier()
    pltpu.sync_copy(acc.at[pl.ds(sid * RPT, RPT)],
                    out.at[cid, pl.ds(sid * RPT, RPT)])

    @pl.when(sid == 0)
    def _():
        pltpu.sync_copy(acc.at[pl.ds(RPT * NS, TAIL)],
                        out.at[cid, pl.ds(RPT * NS, TAIL)])


@functools.lru_cache(maxsize=None)
def _sc_scatter_kernel():
    return pl.kernel(
        _sc_scatter_body,
        out_type=jax.ShapeDtypeStruct((NC, N_NODES, W128), jnp.float32),
        mesh=_mesh(),
        scratch_types=[
            pltpu.VMEM_SHARED((N_NODES, W128), jnp.float32),
            pltpu.VMEM((4, KB), jnp.int32),
            pltpu.VMEM((4, KB), jnp.int32),
            pltpu.VMEM((2, PB, W128), jnp.float32),
            pltpu.VMEM((2, KB, W128), jnp.float32),
            pltpu.SemaphoreType.DMA,
            pltpu.SemaphoreType.DMA,
            pltpu.SemaphoreType.DMA,
            pltpu.SemaphoreType.DMA,
        ],
    )


def _sc_scatter(edges, src1, dst1, zrows):
    return _sc_scatter_kernel()(edges, src1, dst1, zrows)


def _rowvec(v):
    return v.reshape(1, -1)


def _rowvec2(v):
    return jnp.concatenate([v, v]).reshape(1, -1)


def _blkdiag(w):
    z = jnp.zeros_like(w)
    return jnp.block([[w, z], [z, w]])


def _lnp(layer):
    return [_rowvec(layer["ln_g"]), _rowvec(layer["ln_b"])]


def _lnp2(layer):
    return [_rowvec2(layer["ln_g"]), _rowvec2(layer["ln_b"])]


def kernel(x, graph, params):
    src = graph[0]
    dst = graph[1]
    srcI = src.reshape(NW, NB, KB)
    dstI = dst.reshape(NW, NB, KB)
    ne = params["node_encoder"]
    ee = params["edge_encoder"]
    cells = params["cells"]
    ol = params["output_layer"]

    xp = jnp.pad(x, ((0, 0), (0, 5)))
    wn1 = jnp.pad(ne[0]["W"], ((0, 5), (0, 0)))
    we1 = ee[0]["W"]
    wea = jnp.pad(we1[:3], ((0, 5), (0, 0)))
    web = jnp.pad(we1[3:], ((0, 5), (0, 0)))

    half = (jnp.arange(W128) >= F).astype(jnp.float32)
    a2 = jnp.stack([(1.0 - half) / F, half / F], axis=1)
    m2b = jnp.stack([1.0 - half, half], axis=0)

    prep_args = ([wn1, _rowvec(ne[0]["b"])] + _lnp(ne[0]) +
                 [ne[1]["W"], _rowvec(ne[1]["b"])] + _lnp(ne[1]) +
                 [wea, _rowvec(ee[0]["b"]), web])
    nodes, u = _node_prep(xp, prep_args)

    pre = _sc_gather(u, srcI, dstI)
    enc_args = ([a2, m2b] + _lnp2(ee[0]) +
                [_blkdiag(ee[1]["W"]), _rowvec2(ee[1]["b"])] + _lnp2(ee[1]))
    edges = _edge_encode(pre, enc_args)

    zrows = jnp.zeros((N_NODES, W128), jnp.float32)
    for cell in cells:
        nn0, nn1 = cell["node_network"]
        en0, en1 = cell["edge_network"]
        pmsg = _sc_scatter(edges, src, dst, zrows)
        cell_args = ([nn0["W"][:F], nn0["W"][F:], _rowvec(nn0["b"])] +
                     _lnp(nn0) +
                     [nn1["W"], _rowvec(nn1["b"])] + _lnp(nn1) +
                     [en0["W"][:F], _rowvec(en0["b"]), en0["W"][F:2 * F]])
        nodes, u = _node_cell(nodes, pmsg, cell_args)
        pre = _sc_gather(u, srcI, dstI)
        edge_args = ([a2, m2b, _blkdiag(en0["W"][2 * F:])] + _lnp2(en0) +
                     [_blkdiag(en1["W"]), _rowvec2(en1["b"])] + _lnp2(en1))
        edges = _edge_cell(pre, edges, edge_args)

    w2p = jnp.pad(ol[1]["W"], ((0, 0), (0, 128 - EMB)))
    b2p = jnp.pad(ol[1]["b"], ((0, 128 - EMB)))
    head_args = ([ol[0]["W"], _rowvec(ol[0]["b"])] + _lnp(ol[0]) +
                 [w2p, _rowvec(b2p)])
    embp = _out_head(nodes, head_args)
    return embp[:, :EMB], nodes, edges.reshape(N_EDGES, F)
